# Initial kernel scaffold; baseline (speedup 1.0000x reference)
import jax
import jax.numpy as jnp
from jax.experimental import pallas as pl


def kernel(positions, lattice_values, random_shift_per_level, anneal_window):
    raise NotImplementedError("write your pallas kernel here")


# speedup vs baseline: 4.9610x; 4.9610x over previous
import functools

import numpy as np
import jax
import jax.numpy as jnp
from jax import lax
from jax.experimental import pallas as pl
from jax.experimental.pallas import tpu as pltpu
from jax.experimental.pallas import tpu_sc as plsc

_HASH_MUL = np.uint32(2531011)


def _tc_math_body(shift_ref, mult_ref, anneal_ref, pos_ref, idx_ref, bary_ref,
                  *, cap):
    l = pl.program_id(0)

    cf = [(pos_ref[q] + shift_ref[l, q]) * mult_ref[l, q] for q in range(3)]

    s2 = cf[2]
    s1 = cf[1] + s2
    s0 = cf[0] + s1
    e = [s0, s1 - cf[0], s2 - 2.0 * cf[1], -3.0 * cf[2]]

    rem0 = []
    for i in range(4):
        v = e[i] * 0.25
        up = jnp.ceil(v) * 4.0
        down = jnp.floor(v) * 4.0
        rem0.append(jnp.where(up - e[i] < e[i] - down, up, down))
    sum_val = jnp.round((rem0[0] + rem0[1] + rem0[2] + rem0[3]) * 0.25)
    sum_val = sum_val.astype(jnp.int32)

    diff = [e[i] - rem0[i] for i in range(4)]
    rank = []
    for i in range(4):
        r = sum_val
        for j in range(4):
            if j > i:
                r = r + (diff[j] > diff[i]).astype(jnp.int32)
            elif j < i:
                r = r + (diff[j] >= diff[i]).astype(jnp.int32)
        rank.append(r)

    for i in range(4):
        lt = rank[i] < 0
        gt = rank[i] > 3
        rem0[i] = jnp.where(lt, rem0[i] + 4.0,
                            jnp.where(gt, rem0[i] - 4.0, rem0[i]))
        rank[i] = jnp.where(lt, rank[i] + 4,
                            jnp.where(gt, rank[i] - 4, rank[i]))

    delta = [(e[i] - rem0[i]) * 0.25 for i in range(4)]
    eps = []
    for k in range(4):
        acc = jnp.where(rank[0] == k, delta[0], 0.0)
        for i in range(1, 4):
            acc = acc + jnp.where(rank[i] == k, delta[i], 0.0)
        eps.append(acc)
    bary = [1.0 + eps[3] - eps[0],
            eps[2] - eps[3],
            eps[1] - eps[2],
            eps[0] - eps[1]]

    aw = anneal_ref[l]
    rem0i = [rem0[q].astype(jnp.int32) for q in range(3)]
    lvl_off = l * cap
    for r in range(4):
        h = None
        for q in range(3):
            key = rem0i[q] + r - jnp.where(rank[q] > 3 - r, 4, 0)
            ku = key.astype(jnp.uint32)
            h = ku * _HASH_MUL if h is None else (h + ku) * _HASH_MUL
        idx_ref[0, r] = (h & jnp.uint32(cap - 1)).astype(jnp.int32) + lvl_off
        bary_ref[0, r] = bary[r] * aw


def _lattice_meta(pos_t, shift, mult, anneal, *, levels, n_points, cap,
                  block_rows=128):
    nb = n_points // 128
    grid = (levels, nb // block_rows)
    idx, bary = pl.pallas_call(
        functools.partial(_tc_math_body, cap=cap),
        grid=grid,
        in_specs=[
            pl.BlockSpec(memory_space=pltpu.SMEM),
            pl.BlockSpec(memory_space=pltpu.SMEM),
            pl.BlockSpec(memory_space=pltpu.SMEM),
            pl.BlockSpec((3, block_rows, 128), lambda l, b: (0, b, 0)),
        ],
        out_specs=[
            pl.BlockSpec((1, 4, block_rows, 128), lambda l, b: (l, 0, b, 0)),
            pl.BlockSpec((1, 4, block_rows, 128), lambda l, b: (l, 0, b, 0)),
        ],
        out_shape=[
            jax.ShapeDtypeStruct((levels, 4, nb, 128), jnp.int32),
            jax.ShapeDtypeStruct((levels, 4, nb, 128), jnp.float32),
        ],
    )(shift, mult, anneal, pos_t)
    return idx.reshape(levels, 4, n_points), bary.reshape(levels, 4, n_points)


def _sc_gather(table_rep, idx, *, levels, n_points, nr_feat, chunk=512):
    n_workers = 32
    pw = n_points // n_workers
    steps = (pw // chunk) * levels
    mesh = plsc.VectorSubcoreMesh(core_axis_name="c", subcore_axis_name="s")

    @functools.partial(
        pl.kernel,
        out_type=jax.ShapeDtypeStruct((levels, 4, n_points, nr_feat),
                                      jnp.float32),
        mesh=mesh,
        compiler_params=pltpu.CompilerParams(use_tc_tiling_on_sc=False),
        scratch_types=[
            pltpu.VMEM((4, chunk), jnp.int32),
            pltpu.VMEM((4, chunk, 8), jnp.float32),
            pltpu.SemaphoreType.DMA,
        ],
    )
    def sc_kernel(table_hbm, idx_hbm, gout_hbm, idx_v, feats_v, sem):
        wid = lax.axis_index("s") * 2 + lax.axis_index("c")
        base = wid * pw

        def step_body(i, carry):
            l = i % levels
            n0 = base + (i // levels) * chunk
            pltpu.sync_copy(idx_hbm.at[l, :, pl.ds(n0, chunk)], idx_v)
            copies = []
            for r in range(4):
                for s in range(chunk // 128):
                    copies.append(pltpu.async_copy(
                        table_hbm.at[idx_v.at[r, pl.ds(s * 128, 128)]],
                        feats_v.at[r, pl.ds(s * 128, 128)],
                        sem,
                    ))
            for cp in copies:
                cp.wait()
            pltpu.sync_copy(feats_v.at[:, :, pl.ds(0, nr_feat)],
                            gout_hbm.at[l, :, pl.ds(n0, chunk)])
            return carry

        lax.fori_loop(0, steps, step_body, 0)

    return sc_kernel(table_rep, idx)


def _tc_combine_body(gout_ref, bary_ref, out_ref):
    acc = None
    for r in range(4):
        b = bary_ref[0, r]
        b2 = jnp.stack([b, b], axis=-1).reshape(b.shape[0], 2 * b.shape[1])
        term = gout_ref[0, r] * b2
        acc = term if acc is None else acc + term
    out_ref[0] = acc


def _combine(gout, bary, *, levels, n_points, nr_feat, block_rows=128):
    nb = n_points // 128
    grid = (levels, nb // block_rows)
    out_t = pl.pallas_call(
        _tc_combine_body,
        grid=grid,
        in_specs=[
            pl.BlockSpec((1, 4, block_rows, 128 * nr_feat),
                         lambda l, b: (l, 0, b, 0)),
            pl.BlockSpec((1, 4, block_rows, 128), lambda l, b: (l, 0, b, 0)),
        ],
        out_specs=pl.BlockSpec((1, block_rows, 128 * nr_feat),
                               lambda l, b: (l, b, 0)),
        out_shape=jax.ShapeDtypeStruct((levels, nb, 128 * nr_feat),
                                       jnp.float32),
    )(gout, bary)
    return out_t


def kernel(positions, lattice_values, random_shift_per_level, anneal_window):
    n_points, d = positions.shape
    levels, cap, nr_feat = lattice_values.shape
    assert d == 3 and cap & (cap - 1) == 0

    scale = np.asarray([2.0 ** (0.5 * l) for l in range(levels)], np.float32)
    i_arr = np.arange(d, dtype=np.float32)
    inv_std = np.float32(1.0) / np.sqrt((i_arr + 1.0) * (i_arr + 2.0),
                                        dtype=np.float32)
    mult = jnp.asarray(scale[:, None] * np.float32(d + 1) * inv_std[None, :])

    pos_t = positions.T.reshape(3, n_points // 128, 128)
    idx, bary = _lattice_meta(pos_t, random_shift_per_level, mult,
                              anneal_window, levels=levels, n_points=n_points,
                              cap=cap)
    table_rep = jnp.tile(lattice_values.reshape(levels * cap, nr_feat),
                         (1, 8 // nr_feat))
    gout = _sc_gather(table_rep, idx, levels=levels, n_points=n_points,
                      nr_feat=nr_feat)
    gout4 = gout.reshape(levels, 4, n_points // 128, 128 * nr_feat)
    bary4 = bary.reshape(levels, 4, n_points // 128, 128)
    out_t = _combine(gout4, bary4, levels=levels, n_points=n_points,
                     nr_feat=nr_feat)
    out = out_t.reshape(levels, n_points, nr_feat)
    return out.transpose(1, 0, 2).reshape(n_points, levels * nr_feat)


# speedup vs baseline: 42.2572x; 8.5179x over previous
import functools

import numpy as np
import jax
import jax.numpy as jnp
from jax import lax
from jax.experimental import pallas as pl
from jax.experimental.pallas import tpu as pltpu
from jax.experimental.pallas import tpu_sc as plsc

_HASH_MUL = np.uint32(2531011)


def _tc_math_body(shift_ref, mult_ref, anneal_ref, pos_ref, idx_ref, bary_ref,
                  *, cap):
    l = pl.program_id(0)

    cf = [(pos_ref[q] + shift_ref[l, q]) * mult_ref[l, q] for q in range(3)]

    s2 = cf[2]
    s1 = cf[1] + s2
    s0 = cf[0] + s1
    e = [s0, s1 - cf[0], s2 - 2.0 * cf[1], -3.0 * cf[2]]

    rem0 = []
    for i in range(4):
        v = e[i] * 0.25
        up = jnp.ceil(v) * 4.0
        down = jnp.floor(v) * 4.0
        rem0.append(jnp.where(up - e[i] < e[i] - down, up, down))
    sum_val = jnp.round((rem0[0] + rem0[1] + rem0[2] + rem0[3]) * 0.25)
    sum_val = sum_val.astype(jnp.int32)

    diff = [e[i] - rem0[i] for i in range(4)]
    rank = []
    for i in range(4):
        r = sum_val
        for j in range(4):
            if j > i:
                r = r + (diff[j] > diff[i]).astype(jnp.int32)
            elif j < i:
                r = r + (diff[j] >= diff[i]).astype(jnp.int32)
        rank.append(r)

    for i in range(4):
        lt = rank[i] < 0
        gt = rank[i] > 3
        rem0[i] = jnp.where(lt, rem0[i] + 4.0,
                            jnp.where(gt, rem0[i] - 4.0, rem0[i]))
        rank[i] = jnp.where(lt, rank[i] + 4,
                            jnp.where(gt, rank[i] - 4, rank[i]))

    delta = [(e[i] - rem0[i]) * 0.25 for i in range(4)]
    eps = []
    for k in range(4):
        acc = jnp.where(rank[0] == k, delta[0], 0.0)
        for i in range(1, 4):
            acc = acc + jnp.where(rank[i] == k, delta[i], 0.0)
        eps.append(acc)
    bary = [1.0 + eps[3] - eps[0],
            eps[2] - eps[3],
            eps[1] - eps[2],
            eps[0] - eps[1]]

    aw = anneal_ref[l]
    rem0i = [rem0[q].astype(jnp.int32) for q in range(3)]
    lvl_off = l * cap
    for r in range(4):
        h = None
        for q in range(3):
            key = rem0i[q] + r - jnp.where(rank[q] > 3 - r, 4, 0)
            ku = key.astype(jnp.uint32)
            h = ku * _HASH_MUL if h is None else (h + ku) * _HASH_MUL
        idx_ref[0, r] = (h & jnp.uint32(cap - 1)).astype(jnp.int32) + lvl_off
        bary_ref[0, r] = bary[r] * aw


def _lattice_meta(pos_t, shift, mult, anneal, *, levels, n_points, cap,
                  block_rows=128):
    nb = n_points // 128
    grid = (levels, nb // block_rows)
    idx, bary = pl.pallas_call(
        functools.partial(_tc_math_body, cap=cap),
        grid=grid,
        in_specs=[
            pl.BlockSpec(memory_space=pltpu.SMEM),
            pl.BlockSpec(memory_space=pltpu.SMEM),
            pl.BlockSpec(memory_space=pltpu.SMEM),
            pl.BlockSpec((3, block_rows, 128), lambda l, b: (0, b, 0)),
        ],
        out_specs=[
            pl.BlockSpec((1, 4, block_rows, 128), lambda l, b: (l, 0, b, 0)),
            pl.BlockSpec((1, 4, block_rows, 128), lambda l, b: (l, 0, b, 0)),
        ],
        out_shape=[
            jax.ShapeDtypeStruct((levels, 4, nb, 128), jnp.int32),
            jax.ShapeDtypeStruct((levels, 4, nb, 128), jnp.float32),
        ],
    )(shift, mult, anneal, pos_t)
    return idx.reshape(levels, 4, n_points), bary.reshape(levels, 4, n_points)


def _sc_gather_combine(table_rep, idx, bary, *, levels, n_points, nr_feat,
                       chunk=512):
    n_workers = 32
    pw = n_points // n_workers
    lf = levels * nr_feat
    mesh = plsc.VectorSubcoreMesh(core_axis_name="c", subcore_axis_name="s")

    @functools.partial(
        pl.kernel,
        out_type=jax.ShapeDtypeStruct((lf, n_points), jnp.float32),
        mesh=mesh,
        compiler_params=pltpu.CompilerParams(use_tc_tiling_on_sc=False),
        scratch_types=[
            pltpu.VMEM((4, chunk), jnp.int32),
            pltpu.VMEM((4, chunk), jnp.float32),
            pltpu.VMEM((4, chunk, 8), jnp.float32),
            pltpu.VMEM((4, chunk), jnp.float32),
            pltpu.VMEM((4, chunk), jnp.float32),
            pltpu.VMEM((lf, chunk), jnp.float32),
            pltpu.VMEM_SHARED((16, 2, 4, chunk), jnp.float32),
            pltpu.SemaphoreType.DMA,
        ],
    )
    def sc_kernel(table_hbm, idx_hbm, bary_hbm, out_hbm,
                  idx_v, bary_v, feats_v, f0_v, f1_v, res_v, sh_v, sem):
        sid = lax.axis_index("s")
        wid = sid * 2 + lax.axis_index("c")
        base = wid * pw

        def chunk_body(ci, carry):
            n0 = base + ci * chunk

            def level_body(l, carry2):
                pltpu.sync_copy(idx_hbm.at[l, :, pl.ds(n0, chunk)], idx_v)
                pltpu.sync_copy(bary_hbm.at[l, :, pl.ds(n0, chunk)], bary_v)
                copies = []
                for r in range(4):
                    for s in range(chunk // 128):
                        copies.append(pltpu.async_copy(
                            table_hbm.at[idx_v.at[r, pl.ds(s * 128, 128)]],
                            feats_v.at[r, pl.ds(s * 128, 128)],
                            sem,
                        ))
                for cp in copies:
                    cp.wait()
                pltpu.sync_copy(feats_v.at[:, :, 0], sh_v.at[sid, 0])
                pltpu.sync_copy(feats_v.at[:, :, 1], sh_v.at[sid, 1])
                pltpu.sync_copy(sh_v.at[sid, 0], f0_v)
                pltpu.sync_copy(sh_v.at[sid, 1], f1_v)
                for g in range(chunk // 16):
                    acc0 = jnp.zeros((16,), jnp.float32)
                    acc1 = jnp.zeros((16,), jnp.float32)
                    for r in range(4):
                        w = bary_v[r, pl.ds(g * 16, 16)]
                        acc0 = acc0 + w * f0_v[r, pl.ds(g * 16, 16)]
                        acc1 = acc1 + w * f1_v[r, pl.ds(g * 16, 16)]
                    res_v[2 * l, pl.ds(g * 16, 16)] = acc0
                    res_v[2 * l + 1, pl.ds(g * 16, 16)] = acc1
                return carry2

            lax.fori_loop(0, levels, level_body, 0)
            pltpu.sync_copy(res_v, out_hbm.at[:, pl.ds(n0, chunk)])
            return carry

        lax.fori_loop(0, pw // chunk, chunk_body, 0)

    return sc_kernel(table_rep, idx, bary)


def kernel(positions, lattice_values, random_shift_per_level, anneal_window):
    n_points, d = positions.shape
    levels, cap, nr_feat = lattice_values.shape
    assert d == 3 and cap & (cap - 1) == 0

    scale = np.asarray([2.0 ** (0.5 * l) for l in range(levels)], np.float32)
    i_arr = np.arange(d, dtype=np.float32)
    inv_std = np.float32(1.0) / np.sqrt((i_arr + 1.0) * (i_arr + 2.0),
                                        dtype=np.float32)
    mult = jnp.asarray(scale[:, None] * np.float32(d + 1) * inv_std[None, :])

    pos_t = positions.T.reshape(3, n_points // 128, 128)
    idx, bary = _lattice_meta(pos_t, random_shift_per_level, mult,
                              anneal_window, levels=levels, n_points=n_points,
                              cap=cap)
    table_rep = jnp.tile(lattice_values.reshape(levels * cap, nr_feat),
                         (1, 8 // nr_feat))
    out_t = _sc_gather_combine(table_rep, idx, bary, levels=levels,
                               n_points=n_points, nr_feat=nr_feat)
    return out_t.T
